# baseline (speedup 1.0000x reference)
import functools

import jax
import jax.numpy as jnp
from jax import lax
from jax.experimental import pallas as pl
from jax.experimental.pallas import tpu as pltpu
from jax.experimental.pallas import tpu_sc as plsc

_KEYS = (
    (1832780943, 270669613),
    (64467757, 2916123636),
    (2465931498, 255383827),
)
_GOLDEN = 0x1BD11BDA
_ROTS = ((13, 15, 26, 6), (17, 29, 16, 24))

NUM_POS = 4096


def _i32(x):
    return jnp.int32(x & 0xFFFFFFFF) if isinstance(x, int) else x.astype(jnp.int32)


def _const_i32(v):
    return jnp.asarray(v & 0xFFFFFFFF, dtype=jnp.uint32).astype(jnp.int32)


def _rotl(x, r):
    return lax.shift_left(x, jnp.int32(r)) | lax.shift_right_logical(
        x, jnp.int32(32 - r))


def _threefry_pair(k1, k2, ks2, x0, x1):
    ks = (k1, k2, ks2)
    for i in range(5):
        for r in _ROTS[i % 2]:
            x0 = x0 + x1
            x1 = _rotl(x1, r)
            x1 = x1 ^ x0
        x0 = x0 + ks[(i + 1) % 3]
        x1 = x1 + ks[(i + 2) % 3] + jnp.int32(i + 1)
    return x0, x1


def _sample_body(n_row, nsteps, js_per_step, jsub, pblocks, plane,
                 mask_ref, out_ref, mant_ref, bm_ref, bs_ref):
    step = pl.program_id(0)

    @pl.when(step == 0)
    def _init():
        bm_ref[...] = jnp.full(bm_ref.shape, -1, jnp.int32)
        bs_ref[...] = jnp.zeros(bs_ref.shape, jnp.int32)

    kb1 = _const_i32(_KEYS[0][0])
    kv1 = _const_i32(_KEYS[1][0])
    ka1 = _const_i32(_KEYS[2][0])
    kb2 = _const_i32(_KEYS[0][1])
    kv2 = _const_i32(_KEYS[1][1])
    ka2 = _const_i32(_KEYS[2][1])

    riota = lax.broadcasted_iota(jnp.int32, (jsub, 1), 0)
    lane_tile = lax.broadcasted_iota(jnp.int32, (jsub, plane), 1) * jnp.int32(n_row)
    m_all = mask_ref[...]

    for pb in range(pblocks):
        def body(js, carry):
            carry = list(carry)
            shift = (jnp.int32(js_per_step) - js) & jnp.int32(js_per_step - 1)
            m = pltpu.roll(m_all, shift, axis=1)[:, 0:1]
            k1 = jnp.where(m == 0, kb1, jnp.where(m == 1, kv1, ka1))
            k2 = jnp.where(m == 0, kb2, jnp.where(m == 1, kv2, ka2))
            ks2 = (k1 ^ k2) ^ _const_i32(_GOLDEN)
            sg = step * js_per_step + js
            sbase = pb * (plane * n_row) + sg * jsub
            x1 = lane_tile + (k2 + riota + sbase)
            x0, x1 = _threefry_pair(k1, k2, ks2, k1, x1)
            mant = lax.shift_right_logical(x0 ^ x1, jnp.int32(9))
            for c in range(3):
                bm = carry[2 * c]
                bs = carry[2 * c + 1]
                mvld = jnp.where(m == c, mant, jnp.int32(-1))
                upd = mvld > bm
                carry[2 * c] = jnp.where(upd, mvld, bm)
                carry[2 * c + 1] = jnp.where(upd, sg, bs)
            return tuple(carry)

        carry = []
        for c in range(3):
            carry.append(bm_ref[pl.ds(c * jsub, jsub),
                                pl.ds(pb * plane, plane)])
            carry.append(bs_ref[pl.ds(c * jsub, jsub),
                                pl.ds(pb * plane, plane)])
        carry = lax.fori_loop(0, js_per_step, body, tuple(carry), unroll=8)
        for c in range(3):
            bm_ref[pl.ds(c * jsub, jsub), pl.ds(pb * plane, plane)] = \
                carry[2 * c]
            bs_ref[pl.ds(c * jsub, jsub), pl.ds(pb * plane, plane)] = \
                carry[2 * c + 1]

    @pl.when(step == nsteps - 1)
    def _finalize():
        rio = lax.broadcasted_iota(jnp.int32, (jsub, plane), 0)
        for pb in range(pblocks):
            for c in range(3):
                am = bm_ref[pl.ds(c * jsub, jsub), pl.ds(pb * plane, plane)]
                asg = bs_ref[pl.ds(c * jsub, jsub), pl.ds(pb * plane, plane)]
                ar = rio
                rows = jsub
                while rows > 1:
                    rows //= 2
                    bm2 = am[rows:2 * rows]
                    bsg = asg[rows:2 * rows]
                    br = ar[rows:2 * rows]
                    am, asg, ar = am[:rows], asg[:rows], ar[:rows]
                    tk = (bm2 > am) | ((bm2 == am) & (
                        (bsg < asg) | ((bsg == asg) & (br < ar))))
                    am = jnp.where(tk, bm2, am)
                    asg = jnp.where(tk, bsg, asg)
                    ar = jnp.where(tk, br, ar)
                sl = (pl.ds(c, 1), pl.ds(pb * plane, plane))
                out_ref[sl] = asg * jnp.int32(jsub) + ar
                mant_ref[sl] = am


def _sample_indices(mask_head, j_total, num_pairs, n_row=None, jsub=16,
                    js_per_step=128, plane=512, interpret=False):
    if n_row is None:
        n_row = j_total
    nsteps = j_total // (jsub * js_per_step)
    pblocks = num_pairs // plane
    mask_t = mask_head.reshape(j_total // jsub, jsub).T
    body = functools.partial(_sample_body, n_row, nsteps, js_per_step, jsub,
                             pblocks, plane)
    out, mant = pl.pallas_call(
        body,
        grid=(nsteps,),
        in_specs=[pl.BlockSpec((jsub, js_per_step), lambda s: (0, s))],
        out_specs=[pl.BlockSpec((8, num_pairs), lambda s: (0, 0)),
                   pl.BlockSpec((8, num_pairs), lambda s: (0, 0))],
        out_shape=[jax.ShapeDtypeStruct((8, num_pairs), jnp.int32),
                   jax.ShapeDtypeStruct((8, num_pairs), jnp.int32)],
        scratch_shapes=[pltpu.VMEM((3 * jsub, num_pairs), jnp.int32),
                        pltpu.VMEM((3 * jsub, num_pairs), jnp.int32)],
        interpret=interpret,
    )(mask_t)
    return out, mant


_J_TC = 178176


def _sc_sample(mask_hbm, n_total, j0, nj, num_p):
    info = plsc.get_sparse_core_info()
    nc, ns = info.num_cores, info.num_subcores
    nw = nc * ns
    ppw = num_p // nw
    jc = 2048
    nch = nj // jc
    nvec = jc // 16
    mesh = plsc.VectorSubcoreMesh(core_axis_name="c", subcore_axis_name="s")

    kb1 = _const_i32(_KEYS[0][0])
    kv1 = _const_i32(_KEYS[1][0])
    ka1 = _const_i32(_KEYS[2][0])
    kb2 = _const_i32(_KEYS[0][1])
    kv2 = _const_i32(_KEYS[1][1])
    ka2 = _const_i32(_KEYS[2][1])

    @functools.partial(
        pl.kernel,
        mesh=mesh,
        compiler_params=pltpu.CompilerParams(use_tc_tiling_on_sc=False),
        out_type=jax.ShapeDtypeStruct((nw, 6 * ppw), jnp.int32),
        scratch_types=[
            pltpu.VMEM((jc,), jnp.int32),
            pltpu.VMEM((jc,), jnp.int32),
            pltpu.VMEM((jc,), jnp.int32),
            pltpu.VMEM((jc,), jnp.int32),
            pltpu.VMEM((ppw, 6, 16), jnp.int32),
            pltpu.VMEM((6 * ppw,), jnp.int32),
        ],
    )
    def k(mask_ref, out_hbm, mv, k1a, k2a, ks2a, st, outv):
        wid = lax.axis_index("s") * nc + lax.axis_index("c")
        pw0 = wid * ppw
        lanes = lax.iota(jnp.int32, 16)
        neg1 = jnp.full((16,), -1, jnp.int32)
        zero = jnp.zeros((16,), jnp.int32)

        def initp(p, _):
            for c in range(3):
                st[p, c, :] = neg1
                st[p, 3 + c, :] = zero
            return 0

        lax.fori_loop(0, ppw, initp, 0)

        def chunk_body(ch, _):
            pltpu.sync_copy(
                mask_ref.at[pl.ds(pl.multiple_of(j0 + ch * jc, jc), jc)], mv)

            def keys_body(jv, _):
                off = pl.ds(pl.multiple_of(jv * 16, 16), 16)
                m = mv[off]
                k1a[off] = jnp.where(m == 0, kb1, jnp.where(m == 1, kv1, ka1))
                k2a[off] = jnp.where(m == 0, kb2, jnp.where(m == 1, kv2, ka2))
                ks2a[off] = (k1a[off] ^ k2a[off]) ^ _const_i32(_GOLDEN)
                return 0

            lax.fori_loop(0, nvec, keys_body, 0)

            def pq_body(pq, _):
                state = []
                for q in range(4):
                    for r in range(6):
                        state.append(st[pq * 4 + q, r, :])

                def jv_body(jv, carry):
                    carry = list(carry)
                    off = pl.ds(pl.multiple_of(jv * 16, 16), 16)
                    m = mv[off]
                    k1v = k1a[off]
                    k2v = k2a[off]
                    ks2v = ks2a[off]
                    v0 = m == 0
                    v1 = m == 1
                    v2 = m == 2
                    jvg = j0 // 16 + ch * nvec + jv
                    for q in range(4):
                        pg = pw0 + pq * 4 + q
                        base = pg * n_total + j0 + ch * jc + jv * 16
                        x1 = k2v + (lanes + base)
                        x0 = k1v
                        x0, x1 = _threefry_pair(k1v, k2v, ks2v, x0, x1)
                        mant = lax.shift_right_logical(x0 ^ x1, jnp.int32(9))
                        for c, vc in enumerate((v0, v1, v2)):
                            bm = carry[q * 6 + c]
                            bj = carry[q * 6 + 3 + c]
                            mvld = jnp.where(vc, mant, jnp.int32(-1))
                            upd = mvld > bm
                            carry[q * 6 + c] = jnp.where(upd, mvld, bm)
                            carry[q * 6 + 3 + c] = jnp.where(upd, jvg, bj)
                    return tuple(carry)

                state = lax.fori_loop(0, nvec, jv_body, tuple(state))
                for q in range(4):
                    for r in range(6):
                        st[pq * 4 + q, r, :] = state[q * 6 + r]
                return 0

            lax.fori_loop(0, ppw // 4, pq_body, 0)
            return 0

        lax.fori_loop(0, nch, chunk_body, 0)

        big = jnp.int32(0x7FFFFFFF)

        def fin_group(g, _):
            for c in range(3):
                acc = zero
                accm = zero
                for t in range(16):
                    p = g * 16 + t
                    best = jnp.int32(-1)
                    bestj = big
                    bmv = st[p, c, :]
                    bjv = st[p, 3 + c, :]
                    for l in range(16):
                        m_l = bmv[l]
                        j_l = bjv[l] * 16 + l
                        take = (m_l > best) | ((m_l == best) & (j_l < bestj))
                        best = jnp.where(take, m_l, best)
                        bestj = jnp.where(take, j_l, bestj)
                    acc = jnp.where(lanes == t, bestj, acc)
                    accm = jnp.where(lanes == t, best, accm)
                outv[pl.ds(pl.multiple_of(c * ppw + g * 16, 16), 16)] = acc
                outv[pl.ds(pl.multiple_of(3 * ppw + c * ppw + g * 16, 16),
                           16)] = accm
            return 0

        lax.fori_loop(0, ppw // 16, fin_group, 0)
        pltpu.sync_copy(outv, out_hbm.at[wid])

    out = k(mask_hbm)
    o = out.reshape(nw, 2, 3, ppw)
    idx = o[:, 0].transpose(1, 0, 2).reshape(3, num_p)
    mant = o[:, 1].transpose(1, 0, 2).reshape(3, num_p)
    return idx, mant


def _sc_gather_loss(ft, idx_tc, mant_tc, idx_sc, mant_sc):
    info = plsc.get_sparse_core_info()
    nc, ns = info.num_cores, info.num_subcores
    nw = nc * ns
    bpw = NUM_POS // nw
    d = ft.shape[1]
    mesh = plsc.VectorSubcoreMesh(core_axis_name="c", subcore_axis_name="s")

    @functools.partial(
        pl.kernel,
        mesh=mesh,
        compiler_params=pltpu.CompilerParams(use_tc_tiling_on_sc=False),
        out_type=jax.ShapeDtypeStruct((nw, 16), jnp.float32),
        scratch_types=[
            pltpu.VMEM((bpw,), jnp.int32),
            pltpu.VMEM((bpw,), jnp.int32),
            pltpu.VMEM((bpw,), jnp.int32),
            pltpu.VMEM((bpw,), jnp.int32),
            pltpu.VMEM((bpw,), jnp.int32),
            pltpu.VMEM((bpw,), jnp.int32),
            pltpu.VMEM((bpw, d), jnp.float32),
            pltpu.VMEM((bpw, d), jnp.float32),
            pltpu.VMEM((bpw, d), jnp.float32),
            pltpu.VMEM((16,), jnp.float32),
            pltpu.SemaphoreType.DMA,
            pltpu.SemaphoreType.DMA,
            pltpu.SemaphoreType.DMA,
        ],
    )
    def k(ft_hbm, itc_hbm, mtc_hbm, isc_hbm, msc_hbm, out_hbm,
          i0, i1, i2, sidx, smt, sms, r0, r1, r2, acc_v, s0, s1, s2):
        wid = lax.axis_index("s") * nc + lax.axis_index("c")
        base = wid * bpw
        idx_bufs = (i0, i1, i2)
        row_bufs = (r0, r1, r2)
        sems = (s0, s1, s2)
        copies = []
        for c in range(3):
            sl = pl.ds(c * NUM_POS + base, bpw)
            pltpu.sync_copy(itc_hbm.at[sl], idx_bufs[c])
            pltpu.sync_copy(mtc_hbm.at[sl], smt)
            pltpu.sync_copy(isc_hbm.at[sl], sidx)
            pltpu.sync_copy(msc_hbm.at[sl], sms)
            for g in range(bpw // 16):
                off = pl.ds(g * 16, 16)
                take_sc = sms[off] > smt[off]
                idx_bufs[c][off] = jnp.where(take_sc, sidx[off],
                                             idx_bufs[c][off])
            cp = pltpu.async_copy(ft_hbm.at[idx_bufs[c]], row_bufs[c], sems[c])
            copies.append(cp)
        for cp in copies:
            cp.wait()

        half = jnp.float32(0.5)
        one = jnp.float32(1.0)

        def phi(dv):
            ad = jnp.abs(dv)
            return jnp.where(ad < one, half * dv * dv, ad - half)

        def srow(s, acc):
            for cc in range(d // 16):
                sl = pl.ds(cc * 16, 16)
                a = r0[s, sl]
                b = r1[s, sl]
                e = r2[s, sl]
                acc = acc + phi(a - b) + phi(b - e) + phi(a - e)
            return acc

        acc = lax.fori_loop(0, bpw, srow, jnp.zeros((16,), jnp.float32))
        acc_v[...] = acc
        pltpu.sync_copy(acc_v, out_hbm.at[wid])

    return k(ft, idx_tc, mant_tc, idx_sc, mant_sc)


def kernel(features_flat, mask_flat):
    n_total = mask_flat.shape[0]
    mask_i32 = mask_flat.astype(jnp.int32)
    idx_tc, mant_tc = _sample_indices(mask_i32[:_J_TC], _J_TC, NUM_POS,
                                      n_row=n_total)
    idx_sc, mant_sc = _sc_sample(mask_i32, n_total, _J_TC, n_total - _J_TC,
                                 NUM_POS)
    ft = features_flat.T
    parts = _sc_gather_loss(ft, idx_tc[:3].reshape(-1),
                            mant_tc[:3].reshape(-1), idx_sc.reshape(-1),
                            mant_sc.reshape(-1))
    denom = jnp.float32(features_flat.shape[0] * NUM_POS)
    return jnp.sum(parts) / denom


# speedup vs baseline: 1.0779x; 1.0779x over previous
import functools

import jax
import jax.numpy as jnp
from jax import lax
from jax.experimental import pallas as pl
from jax.experimental.pallas import tpu as pltpu
from jax.experimental.pallas import tpu_sc as plsc

_KEYS = (
    (1832780943, 270669613),
    (64467757, 2916123636),
    (2465931498, 255383827),
)
_GOLDEN = 0x1BD11BDA
_ROTS = ((13, 15, 26, 6), (17, 29, 16, 24))

NUM_POS = 4096


def _i32(x):
    return jnp.int32(x & 0xFFFFFFFF) if isinstance(x, int) else x.astype(jnp.int32)


def _const_i32(v):
    return jnp.asarray(v & 0xFFFFFFFF, dtype=jnp.uint32).astype(jnp.int32)


def _rotl(x, r):
    return lax.shift_left(x, jnp.int32(r)) | lax.shift_right_logical(
        x, jnp.int32(32 - r))


def _threefry_pair(k1, k2, ks2, x0, x1):
    ks = (k1, k2, ks2)
    for i in range(5):
        for r in _ROTS[i % 2]:
            x0 = x0 + x1
            x1 = _rotl(x1, r)
            x1 = x1 ^ x0
        x0 = x0 + ks[(i + 1) % 3]
        x1 = x1 + ks[(i + 2) % 3] + jnp.int32(i + 1)
    return x0, x1


def _sample_body(n_row, nsteps, js_per_step, jsub, pblocks, plane,
                 mask_ref, out_ref, mant_ref, bm_ref, bs_ref):
    step = pl.program_id(0)

    @pl.when(step == 0)
    def _init():
        bm_ref[...] = jnp.full(bm_ref.shape, -1, jnp.int32)
        bs_ref[...] = jnp.zeros(bs_ref.shape, jnp.int32)

    kb1 = _const_i32(_KEYS[0][0])
    kv1 = _const_i32(_KEYS[1][0])
    ka1 = _const_i32(_KEYS[2][0])
    kb2 = _const_i32(_KEYS[0][1])
    kv2 = _const_i32(_KEYS[1][1])
    ka2 = _const_i32(_KEYS[2][1])

    riota = lax.broadcasted_iota(jnp.int32, (jsub, 1), 0)
    lane_tile = lax.broadcasted_iota(jnp.int32, (jsub, plane), 1) * jnp.int32(n_row)
    m_all = mask_ref[...]

    for pb in range(pblocks):
        def body(js, carry):
            carry = list(carry)
            shift = (jnp.int32(js_per_step) - js) & jnp.int32(js_per_step - 1)
            m = pltpu.roll(m_all, shift, axis=1)[:, 0:1]
            k1 = jnp.where(m == 0, kb1, jnp.where(m == 1, kv1, ka1))
            k2 = jnp.where(m == 0, kb2, jnp.where(m == 1, kv2, ka2))
            ks2 = (k1 ^ k2) ^ _const_i32(_GOLDEN)
            sg = step * js_per_step + js
            sbase = pb * (plane * n_row) + sg * jsub
            x1 = lane_tile + (k2 + riota + sbase)
            x0, x1 = _threefry_pair(k1, k2, ks2, k1, x1)
            mant = lax.shift_right_logical(x0 ^ x1, jnp.int32(9))
            for c in range(3):
                bm = carry[2 * c]
                bs = carry[2 * c + 1]
                mvld = jnp.where(m == c, mant, jnp.int32(-1))
                upd = mvld > bm
                carry[2 * c] = jnp.where(upd, mvld, bm)
                carry[2 * c + 1] = jnp.where(upd, sg, bs)
            return tuple(carry)

        carry = []
        for c in range(3):
            carry.append(bm_ref[pl.ds(c * jsub, jsub),
                                pl.ds(pb * plane, plane)])
            carry.append(bs_ref[pl.ds(c * jsub, jsub),
                                pl.ds(pb * plane, plane)])
        carry = lax.fori_loop(0, js_per_step, body, tuple(carry), unroll=8)
        for c in range(3):
            bm_ref[pl.ds(c * jsub, jsub), pl.ds(pb * plane, plane)] = \
                carry[2 * c]
            bs_ref[pl.ds(c * jsub, jsub), pl.ds(pb * plane, plane)] = \
                carry[2 * c + 1]

    @pl.when(step == nsteps - 1)
    def _finalize():
        rio = lax.broadcasted_iota(jnp.int32, (jsub, plane), 0)
        for pb in range(pblocks):
            for c in range(3):
                am = bm_ref[pl.ds(c * jsub, jsub), pl.ds(pb * plane, plane)]
                asg = bs_ref[pl.ds(c * jsub, jsub), pl.ds(pb * plane, plane)]
                ar = rio
                rows = jsub
                while rows > 1:
                    rows //= 2
                    bm2 = am[rows:2 * rows]
                    bsg = asg[rows:2 * rows]
                    br = ar[rows:2 * rows]
                    am, asg, ar = am[:rows], asg[:rows], ar[:rows]
                    tk = (bm2 > am) | ((bm2 == am) & (
                        (bsg < asg) | ((bsg == asg) & (br < ar))))
                    am = jnp.where(tk, bm2, am)
                    asg = jnp.where(tk, bsg, asg)
                    ar = jnp.where(tk, br, ar)
                sl = (pl.ds(c, 1), pl.ds(pb * plane, plane))
                out_ref[sl] = asg * jnp.int32(jsub) + ar
                mant_ref[sl] = am


def _sample_indices(mask_head, j_total, num_pairs, n_row=None, jsub=16,
                    js_per_step=128, plane=512, interpret=False):
    if n_row is None:
        n_row = j_total
    nsteps = j_total // (jsub * js_per_step)
    pblocks = num_pairs // plane
    mask_t = mask_head.reshape(j_total // jsub, jsub).T
    body = functools.partial(_sample_body, n_row, nsteps, js_per_step, jsub,
                             pblocks, plane)
    out, mant = pl.pallas_call(
        body,
        grid=(nsteps,),
        in_specs=[pl.BlockSpec((jsub, js_per_step), lambda s: (0, s))],
        out_specs=[pl.BlockSpec((8, num_pairs), lambda s: (0, 0)),
                   pl.BlockSpec((8, num_pairs), lambda s: (0, 0))],
        out_shape=[jax.ShapeDtypeStruct((8, num_pairs), jnp.int32),
                   jax.ShapeDtypeStruct((8, num_pairs), jnp.int32)],
        scratch_shapes=[pltpu.VMEM((3 * jsub, num_pairs), jnp.int32),
                        pltpu.VMEM((3 * jsub, num_pairs), jnp.int32)],
        interpret=interpret,
    )(mask_t)
    return out, mant


_J_TC = 184320


def _sc_sample(mask_hbm, n_total, j0, nj, num_p):
    info = plsc.get_sparse_core_info()
    nc, ns = info.num_cores, info.num_subcores
    nw = nc * ns
    ppw = num_p // nw
    jc = 2048
    nch = nj // jc
    nvec = jc // 16
    mesh = plsc.VectorSubcoreMesh(core_axis_name="c", subcore_axis_name="s")

    kb1 = _const_i32(_KEYS[0][0])
    kv1 = _const_i32(_KEYS[1][0])
    ka1 = _const_i32(_KEYS[2][0])
    kb2 = _const_i32(_KEYS[0][1])
    kv2 = _const_i32(_KEYS[1][1])
    ka2 = _const_i32(_KEYS[2][1])

    @functools.partial(
        pl.kernel,
        mesh=mesh,
        compiler_params=pltpu.CompilerParams(use_tc_tiling_on_sc=False),
        out_type=jax.ShapeDtypeStruct((nw, 6 * ppw), jnp.int32),
        scratch_types=[
            pltpu.VMEM((jc,), jnp.int32),
            pltpu.VMEM((jc,), jnp.int32),
            pltpu.VMEM((jc,), jnp.int32),
            pltpu.VMEM((jc,), jnp.int32),
            pltpu.VMEM((ppw, 6, 16), jnp.int32),
            pltpu.VMEM((6 * ppw,), jnp.int32),
        ],
    )
    def k(mask_ref, out_hbm, mv, k1a, k2a, ks2a, st, outv):
        wid = lax.axis_index("s") * nc + lax.axis_index("c")
        pw0 = wid * ppw
        lanes = lax.iota(jnp.int32, 16)
        neg1 = jnp.full((16,), -1, jnp.int32)
        zero = jnp.zeros((16,), jnp.int32)

        def initp(p, _):
            for c in range(3):
                st[p, c, :] = neg1
                st[p, 3 + c, :] = zero
            return 0

        lax.fori_loop(0, ppw, initp, 0)

        def chunk_body(ch, _):
            pltpu.sync_copy(
                mask_ref.at[pl.ds(pl.multiple_of(j0 + ch * jc, jc), jc)], mv)

            def keys_body(jv, _):
                off = pl.ds(pl.multiple_of(jv * 16, 16), 16)
                m = mv[off]
                k1a[off] = jnp.where(m == 0, kb1, jnp.where(m == 1, kv1, ka1))
                k2a[off] = jnp.where(m == 0, kb2, jnp.where(m == 1, kv2, ka2))
                ks2a[off] = (k1a[off] ^ k2a[off]) ^ _const_i32(_GOLDEN)
                return 0

            lax.fori_loop(0, nvec, keys_body, 0)

            def pq_body(pq, _):
                state = []
                for q in range(4):
                    for r in range(6):
                        state.append(st[pq * 4 + q, r, :])

                def jv_body(jv, carry):
                    carry = list(carry)
                    off = pl.ds(pl.multiple_of(jv * 16, 16), 16)
                    m = mv[off]
                    k1v = k1a[off]
                    k2v = k2a[off]
                    ks2v = ks2a[off]
                    v0 = m == 0
                    v1 = m == 1
                    v2 = m == 2
                    jvg = j0 // 16 + ch * nvec + jv
                    for q in range(4):
                        pg = pw0 + pq * 4 + q
                        base = pg * n_total + j0 + ch * jc + jv * 16
                        x1 = k2v + (lanes + base)
                        x0 = k1v
                        x0, x1 = _threefry_pair(k1v, k2v, ks2v, x0, x1)
                        mant = lax.shift_right_logical(x0 ^ x1, jnp.int32(9))
                        for c, vc in enumerate((v0, v1, v2)):
                            bm = carry[q * 6 + c]
                            bj = carry[q * 6 + 3 + c]
                            mvld = jnp.where(vc, mant, jnp.int32(-1))
                            upd = mvld > bm
                            carry[q * 6 + c] = jnp.where(upd, mvld, bm)
                            carry[q * 6 + 3 + c] = jnp.where(upd, jvg, bj)
                    return tuple(carry)

                state = lax.fori_loop(0, nvec, jv_body, tuple(state))
                for q in range(4):
                    for r in range(6):
                        st[pq * 4 + q, r, :] = state[q * 6 + r]
                return 0

            lax.fori_loop(0, ppw // 4, pq_body, 0)
            return 0

        lax.fori_loop(0, nch, chunk_body, 0)

        big = jnp.int32(0x7FFFFFFF)

        def fin_group(g, _):
            for c in range(3):
                acc = zero
                accm = zero
                for t in range(16):
                    p = g * 16 + t
                    best = jnp.int32(-1)
                    bestj = big
                    bmv = st[p, c, :]
                    bjv = st[p, 3 + c, :]
                    for l in range(16):
                        m_l = bmv[l]
                        j_l = bjv[l] * 16 + l
                        take = (m_l > best) | ((m_l == best) & (j_l < bestj))
                        best = jnp.where(take, m_l, best)
                        bestj = jnp.where(take, j_l, bestj)
                    acc = jnp.where(lanes == t, bestj, acc)
                    accm = jnp.where(lanes == t, best, accm)
                outv[pl.ds(pl.multiple_of(c * ppw + g * 16, 16), 16)] = acc
                outv[pl.ds(pl.multiple_of(3 * ppw + c * ppw + g * 16, 16),
                           16)] = accm
            return 0

        lax.fori_loop(0, ppw // 16, fin_group, 0)
        pltpu.sync_copy(outv, out_hbm.at[wid])

    out = k(mask_hbm)
    o = out.reshape(nw, 2, 3, ppw)
    idx = o[:, 0].transpose(1, 0, 2).reshape(3, num_p)
    mant = o[:, 1].transpose(1, 0, 2).reshape(3, num_p)
    return idx, mant


def _sc_gather_loss(ft, idx_tc, mant_tc, idx_sc, mant_sc):
    info = plsc.get_sparse_core_info()
    nc, ns = info.num_cores, info.num_subcores
    nw = nc * ns
    bpw = NUM_POS // nw
    d = ft.shape[1]
    mesh = plsc.VectorSubcoreMesh(core_axis_name="c", subcore_axis_name="s")

    @functools.partial(
        pl.kernel,
        mesh=mesh,
        compiler_params=pltpu.CompilerParams(use_tc_tiling_on_sc=False),
        out_type=jax.ShapeDtypeStruct((nw, 16), jnp.float32),
        scratch_types=[
            pltpu.VMEM((bpw,), jnp.int32),
            pltpu.VMEM((bpw,), jnp.int32),
            pltpu.VMEM((bpw,), jnp.int32),
            pltpu.VMEM((bpw,), jnp.int32),
            pltpu.VMEM((bpw,), jnp.int32),
            pltpu.VMEM((bpw,), jnp.int32),
            pltpu.VMEM((bpw, d), jnp.float32),
            pltpu.VMEM((bpw, d), jnp.float32),
            pltpu.VMEM((bpw, d), jnp.float32),
            pltpu.VMEM((16,), jnp.float32),
            pltpu.SemaphoreType.DMA,
            pltpu.SemaphoreType.DMA,
            pltpu.SemaphoreType.DMA,
        ],
    )
    def k(ft_hbm, itc_hbm, mtc_hbm, isc_hbm, msc_hbm, out_hbm,
          i0, i1, i2, sidx, smt, sms, r0, r1, r2, acc_v, s0, s1, s2):
        wid = lax.axis_index("s") * nc + lax.axis_index("c")
        base = wid * bpw
        idx_bufs = (i0, i1, i2)
        row_bufs = (r0, r1, r2)
        sems = (s0, s1, s2)
        copies = []
        for c in range(3):
            sl = pl.ds(c * NUM_POS + base, bpw)
            pltpu.sync_copy(itc_hbm.at[sl], idx_bufs[c])
            pltpu.sync_copy(mtc_hbm.at[sl], smt)
            pltpu.sync_copy(isc_hbm.at[sl], sidx)
            pltpu.sync_copy(msc_hbm.at[sl], sms)
            for g in range(bpw // 16):
                off = pl.ds(g * 16, 16)
                take_sc = sms[off] > smt[off]
                idx_bufs[c][off] = jnp.where(take_sc, sidx[off],
                                             idx_bufs[c][off])
            cp = pltpu.async_copy(ft_hbm.at[idx_bufs[c]], row_bufs[c], sems[c])
            copies.append(cp)
        for cp in copies:
            cp.wait()

        half = jnp.float32(0.5)
        one = jnp.float32(1.0)

        def phi(dv):
            ad = jnp.abs(dv)
            return jnp.where(ad < one, half * dv * dv, ad - half)

        def srow(s, acc):
            for cc in range(d // 16):
                sl = pl.ds(cc * 16, 16)
                a = r0[s, sl]
                b = r1[s, sl]
                e = r2[s, sl]
                acc = acc + phi(a - b) + phi(b - e) + phi(a - e)
            return acc

        acc = lax.fori_loop(0, bpw, srow, jnp.zeros((16,), jnp.float32))
        acc_v[...] = acc
        pltpu.sync_copy(acc_v, out_hbm.at[wid])

    return k(ft, idx_tc, mant_tc, idx_sc, mant_sc)


def kernel(features_flat, mask_flat):
    n_total = mask_flat.shape[0]
    mask_i32 = mask_flat.astype(jnp.int32)
    idx_tc, mant_tc = _sample_indices(mask_i32[:_J_TC], _J_TC, NUM_POS,
                                      n_row=n_total)
    idx_sc, mant_sc = _sc_sample(mask_i32, n_total, _J_TC, n_total - _J_TC,
                                 NUM_POS)
    ft = features_flat.T
    parts = _sc_gather_loss(ft, idx_tc[:3].reshape(-1),
                            mant_tc[:3].reshape(-1), idx_sc.reshape(-1),
                            mant_sc.reshape(-1))
    denom = jnp.float32(features_flat.shape[0] * NUM_POS)
    return jnp.sum(parts) / denom


# speedup vs baseline: 1.0798x; 1.0018x over previous
import functools

import jax
import jax.numpy as jnp
from jax import lax
from jax.experimental import pallas as pl
from jax.experimental.pallas import tpu as pltpu
from jax.experimental.pallas import tpu_sc as plsc

_KEYS = (
    (1832780943, 270669613),
    (64467757, 2916123636),
    (2465931498, 255383827),
)
_GOLDEN = 0x1BD11BDA
_ROTS = ((13, 15, 26, 6), (17, 29, 16, 24))

NUM_POS = 4096


def _i32(x):
    return jnp.int32(x & 0xFFFFFFFF) if isinstance(x, int) else x.astype(jnp.int32)


def _const_i32(v):
    return jnp.asarray(v & 0xFFFFFFFF, dtype=jnp.uint32).astype(jnp.int32)


def _rotl(x, r):
    return lax.shift_left(x, jnp.int32(r)) | lax.shift_right_logical(
        x, jnp.int32(32 - r))


def _threefry_pair(k1, k2, ks2, x0, x1):
    ks = (k1, k2, ks2)
    for i in range(5):
        for r in _ROTS[i % 2]:
            x0 = x0 + x1
            x1 = _rotl(x1, r)
            x1 = x1 ^ x0
        x0 = x0 + ks[(i + 1) % 3]
        x1 = x1 + ks[(i + 2) % 3] + jnp.int32(i + 1)
    return x0, x1


def _sample_body(n_row, nsteps, js_per_step, jsub, pblocks, plane,
                 mask_ref, out_ref, mant_ref, bm_ref, bs_ref):
    step = pl.program_id(0)

    @pl.when(step == 0)
    def _init():
        bm_ref[...] = jnp.full(bm_ref.shape, -1, jnp.int32)
        bs_ref[...] = jnp.zeros(bs_ref.shape, jnp.int32)

    kb1 = _const_i32(_KEYS[0][0])
    kv1 = _const_i32(_KEYS[1][0])
    ka1 = _const_i32(_KEYS[2][0])
    kb2 = _const_i32(_KEYS[0][1])
    kv2 = _const_i32(_KEYS[1][1])
    ka2 = _const_i32(_KEYS[2][1])

    riota = lax.broadcasted_iota(jnp.int32, (jsub, 1), 0)
    lane_tile = lax.broadcasted_iota(jnp.int32, (jsub, plane), 1) * jnp.int32(n_row)
    m_all = mask_ref[...]

    for pb in range(pblocks):
        def body(js, carry):
            carry = list(carry)
            shift = (jnp.int32(js_per_step) - js) & jnp.int32(js_per_step - 1)
            m = pltpu.roll(m_all, shift, axis=1)[:, 0:1]
            k1 = jnp.where(m == 0, kb1, jnp.where(m == 1, kv1, ka1))
            k2 = jnp.where(m == 0, kb2, jnp.where(m == 1, kv2, ka2))
            ks2 = (k1 ^ k2) ^ _const_i32(_GOLDEN)
            sg = step * js_per_step + js
            sbase = pb * (plane * n_row) + sg * jsub
            x1 = lane_tile + (k2 + riota + sbase)
            x0, x1 = _threefry_pair(k1, k2, ks2, k1, x1)
            mant = lax.shift_right_logical(x0 ^ x1, jnp.int32(9))
            for c in range(3):
                bm = carry[2 * c]
                bs = carry[2 * c + 1]
                mvld = jnp.where(m == c, mant, jnp.int32(-1))
                upd = mvld > bm
                carry[2 * c] = jnp.where(upd, mvld, bm)
                carry[2 * c + 1] = jnp.where(upd, sg, bs)
            return tuple(carry)

        carry = []
        for c in range(3):
            carry.append(bm_ref[pl.ds(c * jsub, jsub),
                                pl.ds(pb * plane, plane)])
            carry.append(bs_ref[pl.ds(c * jsub, jsub),
                                pl.ds(pb * plane, plane)])
        carry = lax.fori_loop(0, js_per_step, body, tuple(carry), unroll=8)
        for c in range(3):
            bm_ref[pl.ds(c * jsub, jsub), pl.ds(pb * plane, plane)] = \
                carry[2 * c]
            bs_ref[pl.ds(c * jsub, jsub), pl.ds(pb * plane, plane)] = \
                carry[2 * c + 1]

    @pl.when(step == nsteps - 1)
    def _finalize():
        rio = lax.broadcasted_iota(jnp.int32, (jsub, plane), 0)
        for pb in range(pblocks):
            for c in range(3):
                am = bm_ref[pl.ds(c * jsub, jsub), pl.ds(pb * plane, plane)]
                asg = bs_ref[pl.ds(c * jsub, jsub), pl.ds(pb * plane, plane)]
                ar = rio
                rows = jsub
                while rows > 1:
                    rows //= 2
                    bm2 = am[rows:2 * rows]
                    bsg = asg[rows:2 * rows]
                    br = ar[rows:2 * rows]
                    am, asg, ar = am[:rows], asg[:rows], ar[:rows]
                    tk = (bm2 > am) | ((bm2 == am) & (
                        (bsg < asg) | ((bsg == asg) & (br < ar))))
                    am = jnp.where(tk, bm2, am)
                    asg = jnp.where(tk, bsg, asg)
                    ar = jnp.where(tk, br, ar)
                sl = (pl.ds(c, 1), pl.ds(pb * plane, plane))
                out_ref[sl] = asg * jnp.int32(jsub) + ar
                mant_ref[sl] = am


def _sample_indices(mask_head, j_total, num_pairs, n_row=None, jsub=16,
                    js_per_step=128, plane=512, interpret=False):
    if n_row is None:
        n_row = j_total
    nsteps = j_total // (jsub * js_per_step)
    pblocks = num_pairs // plane
    mask_t = mask_head.reshape(j_total // jsub, jsub).T
    body = functools.partial(_sample_body, n_row, nsteps, js_per_step, jsub,
                             pblocks, plane)
    out, mant = pl.pallas_call(
        body,
        grid=(nsteps,),
        in_specs=[pl.BlockSpec((jsub, js_per_step), lambda s: (0, s))],
        out_specs=[pl.BlockSpec((8, num_pairs), lambda s: (0, 0)),
                   pl.BlockSpec((8, num_pairs), lambda s: (0, 0))],
        out_shape=[jax.ShapeDtypeStruct((8, num_pairs), jnp.int32),
                   jax.ShapeDtypeStruct((8, num_pairs), jnp.int32)],
        scratch_shapes=[pltpu.VMEM((3 * jsub, num_pairs), jnp.int32),
                        pltpu.VMEM((3 * jsub, num_pairs), jnp.int32)],
        interpret=interpret,
    )(mask_t)
    return out, mant


_J_TC = 186368


def _sc_sample(mask_hbm, n_total, j0, nj, num_p):
    info = plsc.get_sparse_core_info()
    nc, ns = info.num_cores, info.num_subcores
    nw = nc * ns
    ppw = num_p // nw
    jc = 2048
    nch = nj // jc
    nvec = jc // 16
    mesh = plsc.VectorSubcoreMesh(core_axis_name="c", subcore_axis_name="s")

    kb1 = _const_i32(_KEYS[0][0])
    kv1 = _const_i32(_KEYS[1][0])
    ka1 = _const_i32(_KEYS[2][0])
    kb2 = _const_i32(_KEYS[0][1])
    kv2 = _const_i32(_KEYS[1][1])
    ka2 = _const_i32(_KEYS[2][1])

    @functools.partial(
        pl.kernel,
        mesh=mesh,
        compiler_params=pltpu.CompilerParams(use_tc_tiling_on_sc=False),
        out_type=jax.ShapeDtypeStruct((nw, 6 * ppw), jnp.int32),
        scratch_types=[
            pltpu.VMEM((jc,), jnp.int32),
            pltpu.VMEM((jc,), jnp.int32),
            pltpu.VMEM((jc,), jnp.int32),
            pltpu.VMEM((jc,), jnp.int32),
            pltpu.VMEM((ppw, 6, 16), jnp.int32),
            pltpu.VMEM((6 * ppw,), jnp.int32),
        ],
    )
    def k(mask_ref, out_hbm, mv, k1a, k2a, ks2a, st, outv):
        wid = lax.axis_index("s") * nc + lax.axis_index("c")
        pw0 = wid * ppw
        lanes = lax.iota(jnp.int32, 16)
        neg1 = jnp.full((16,), -1, jnp.int32)
        zero = jnp.zeros((16,), jnp.int32)

        def initp(p, _):
            for c in range(3):
                st[p, c, :] = neg1
                st[p, 3 + c, :] = zero
            return 0

        lax.fori_loop(0, ppw, initp, 0)

        def chunk_body(ch, _):
            pltpu.sync_copy(
                mask_ref.at[pl.ds(pl.multiple_of(j0 + ch * jc, jc), jc)], mv)

            def keys_body(jv, _):
                off = pl.ds(pl.multiple_of(jv * 16, 16), 16)
                m = mv[off]
                k1a[off] = jnp.where(m == 0, kb1, jnp.where(m == 1, kv1, ka1))
                k2a[off] = jnp.where(m == 0, kb2, jnp.where(m == 1, kv2, ka2))
                ks2a[off] = (k1a[off] ^ k2a[off]) ^ _const_i32(_GOLDEN)
                return 0

            lax.fori_loop(0, nvec, keys_body, 0)

            def pq_body(pq, _):
                state = []
                for q in range(4):
                    for r in range(6):
                        state.append(st[pq * 4 + q, r, :])

                def jv_body(jv, carry):
                    carry = list(carry)
                    off = pl.ds(pl.multiple_of(jv * 16, 16), 16)
                    m = mv[off]
                    k1v = k1a[off]
                    k2v = k2a[off]
                    ks2v = ks2a[off]
                    v0 = m == 0
                    v1 = m == 1
                    v2 = m == 2
                    jvg = j0 // 16 + ch * nvec + jv
                    for q in range(4):
                        pg = pw0 + pq * 4 + q
                        base = pg * n_total + j0 + ch * jc + jv * 16
                        x1 = k2v + (lanes + base)
                        x0 = k1v
                        x0, x1 = _threefry_pair(k1v, k2v, ks2v, x0, x1)
                        mant = lax.shift_right_logical(x0 ^ x1, jnp.int32(9))
                        for c, vc in enumerate((v0, v1, v2)):
                            bm = carry[q * 6 + c]
                            bj = carry[q * 6 + 3 + c]
                            mvld = jnp.where(vc, mant, jnp.int32(-1))
                            upd = mvld > bm
                            carry[q * 6 + c] = jnp.where(upd, mvld, bm)
                            carry[q * 6 + 3 + c] = jnp.where(upd, jvg, bj)
                    return tuple(carry)

                state = lax.fori_loop(0, nvec, jv_body, tuple(state))
                for q in range(4):
                    for r in range(6):
                        st[pq * 4 + q, r, :] = state[q * 6 + r]
                return 0

            lax.fori_loop(0, ppw // 4, pq_body, 0)
            return 0

        lax.fori_loop(0, nch, chunk_body, 0)

        big = jnp.int32(0x7FFFFFFF)

        def fin_group(g, _):
            for c in range(3):
                acc = zero
                accm = zero
                for t in range(16):
                    p = g * 16 + t
                    best = jnp.int32(-1)
                    bestj = big
                    bmv = st[p, c, :]
                    bjv = st[p, 3 + c, :]
                    for l in range(16):
                        m_l = bmv[l]
                        j_l = bjv[l] * 16 + l
                        take = (m_l > best) | ((m_l == best) & (j_l < bestj))
                        best = jnp.where(take, m_l, best)
                        bestj = jnp.where(take, j_l, bestj)
                    acc = jnp.where(lanes == t, bestj, acc)
                    accm = jnp.where(lanes == t, best, accm)
                outv[pl.ds(pl.multiple_of(c * ppw + g * 16, 16), 16)] = acc
                outv[pl.ds(pl.multiple_of(3 * ppw + c * ppw + g * 16, 16),
                           16)] = accm
            return 0

        lax.fori_loop(0, ppw // 16, fin_group, 0)
        pltpu.sync_copy(outv, out_hbm.at[wid])

    out = k(mask_hbm)
    o = out.reshape(nw, 2, 3, ppw)
    idx = o[:, 0].transpose(1, 0, 2).reshape(3, num_p)
    mant = o[:, 1].transpose(1, 0, 2).reshape(3, num_p)
    return idx, mant


def _sc_gather_loss(ft, idx_tc, mant_tc, idx_sc, mant_sc):
    info = plsc.get_sparse_core_info()
    nc, ns = info.num_cores, info.num_subcores
    nw = nc * ns
    bpw = NUM_POS // nw
    d = ft.shape[1]
    mesh = plsc.VectorSubcoreMesh(core_axis_name="c", subcore_axis_name="s")

    @functools.partial(
        pl.kernel,
        mesh=mesh,
        compiler_params=pltpu.CompilerParams(use_tc_tiling_on_sc=False),
        out_type=jax.ShapeDtypeStruct((nw, 16), jnp.float32),
        scratch_types=[
            pltpu.VMEM((bpw,), jnp.int32),
            pltpu.VMEM((bpw,), jnp.int32),
            pltpu.VMEM((bpw,), jnp.int32),
            pltpu.VMEM((bpw,), jnp.int32),
            pltpu.VMEM((bpw,), jnp.int32),
            pltpu.VMEM((bpw,), jnp.int32),
            pltpu.VMEM((bpw, d), jnp.float32),
            pltpu.VMEM((bpw, d), jnp.float32),
            pltpu.VMEM((bpw, d), jnp.float32),
            pltpu.VMEM((16,), jnp.float32),
            pltpu.SemaphoreType.DMA,
            pltpu.SemaphoreType.DMA,
            pltpu.SemaphoreType.DMA,
        ],
    )
    def k(ft_hbm, itc_hbm, mtc_hbm, isc_hbm, msc_hbm, out_hbm,
          i0, i1, i2, sidx, smt, sms, r0, r1, r2, acc_v, s0, s1, s2):
        wid = lax.axis_index("s") * nc + lax.axis_index("c")
        base = wid * bpw
        idx_bufs = (i0, i1, i2)
        row_bufs = (r0, r1, r2)
        sems = (s0, s1, s2)
        copies = []
        for c in range(3):
            sl = pl.ds(c * NUM_POS + base, bpw)
            pltpu.sync_copy(itc_hbm.at[sl], idx_bufs[c])
            pltpu.sync_copy(mtc_hbm.at[sl], smt)
            pltpu.sync_copy(isc_hbm.at[sl], sidx)
            pltpu.sync_copy(msc_hbm.at[sl], sms)
            for g in range(bpw // 16):
                off = pl.ds(g * 16, 16)
                take_sc = sms[off] > smt[off]
                idx_bufs[c][off] = jnp.where(take_sc, sidx[off],
                                             idx_bufs[c][off])
            cp = pltpu.async_copy(ft_hbm.at[idx_bufs[c]], row_bufs[c], sems[c])
            copies.append(cp)
        for cp in copies:
            cp.wait()

        half = jnp.float32(0.5)
        one = jnp.float32(1.0)

        def phi(dv):
            ad = jnp.abs(dv)
            return jnp.where(ad < one, half * dv * dv, ad - half)

        def srow(s, acc):
            for cc in range(d // 16):
                sl = pl.ds(cc * 16, 16)
                a = r0[s, sl]
                b = r1[s, sl]
                e = r2[s, sl]
                acc = acc + phi(a - b) + phi(b - e) + phi(a - e)
            return acc

        acc = lax.fori_loop(0, bpw, srow, jnp.zeros((16,), jnp.float32))
        acc_v[...] = acc
        pltpu.sync_copy(acc_v, out_hbm.at[wid])

    return k(ft, idx_tc, mant_tc, idx_sc, mant_sc)


def kernel(features_flat, mask_flat):
    n_total = mask_flat.shape[0]
    mask_i32 = mask_flat.astype(jnp.int32)
    idx_tc, mant_tc = _sample_indices(mask_i32[:_J_TC], _J_TC, NUM_POS,
                                      n_row=n_total)
    idx_sc, mant_sc = _sc_sample(mask_i32, n_total, _J_TC, n_total - _J_TC,
                                 NUM_POS)
    ft = features_flat.T
    parts = _sc_gather_loss(ft, idx_tc[:3].reshape(-1),
                            mant_tc[:3].reshape(-1), idx_sc.reshape(-1),
                            mant_sc.reshape(-1))
    denom = jnp.float32(features_flat.shape[0] * NUM_POS)
    return jnp.sum(parts) / denom


# speedup vs baseline: 1.0798x; 1.0000x over previous
import functools

import jax
import jax.numpy as jnp
from jax import lax
from jax.experimental import pallas as pl
from jax.experimental.pallas import tpu as pltpu
from jax.experimental.pallas import tpu_sc as plsc

_KEYS = (
    (1832780943, 270669613),
    (64467757, 2916123636),
    (2465931498, 255383827),
)
_GOLDEN = 0x1BD11BDA
_ROTS = ((13, 15, 26, 6), (17, 29, 16, 24))

NUM_POS = 4096


def _const_i32(v):
    return jnp.asarray(v & 0xFFFFFFFF, dtype=jnp.uint32).astype(jnp.int32)


def _rotl(x, r):
    return lax.shift_left(x, jnp.int32(r)) | lax.shift_right_logical(
        x, jnp.int32(32 - r))


def _threefry_pair(k1, k2, ks2, x0, x1):
    ks = (k1, k2, ks2)
    for i in range(5):
        for r in _ROTS[i % 2]:
            x0 = x0 + x1
            x1 = _rotl(x1, r)
            x1 = x1 ^ x0
        x0 = x0 + ks[(i + 1) % 3]
        x1 = x1 + ks[(i + 2) % 3] + jnp.int32(i + 1)
    return x0, x1


def _sample_body(n_row, nsteps, js_per_step, jsub, pblocks, plane,
                 mask_ref, out_ref, mant_ref, bm_ref, bs_ref):
    step = pl.program_id(0)

    @pl.when(step == 0)
    def _init():
        bm_ref[...] = jnp.full(bm_ref.shape, -1, jnp.int32)
        bs_ref[...] = jnp.zeros(bs_ref.shape, jnp.int32)

    kb1 = _const_i32(_KEYS[0][0])
    kv1 = _const_i32(_KEYS[1][0])
    ka1 = _const_i32(_KEYS[2][0])
    kb2 = _const_i32(_KEYS[0][1])
    kv2 = _const_i32(_KEYS[1][1])
    ka2 = _const_i32(_KEYS[2][1])

    riota = lax.broadcasted_iota(jnp.int32, (jsub, 1), 0)
    lane_tile = lax.broadcasted_iota(jnp.int32, (jsub, plane), 1) * jnp.int32(n_row)
    m_all = mask_ref[...]

    for pb in range(pblocks):
        def body(js, carry):
            carry = list(carry)
            shift = (jnp.int32(js_per_step) - js) & jnp.int32(js_per_step - 1)
            m = pltpu.roll(m_all, shift, axis=1)[:, 0:1]
            k1 = jnp.where(m == 0, kb1, jnp.where(m == 1, kv1, ka1))
            k2 = jnp.where(m == 0, kb2, jnp.where(m == 1, kv2, ka2))
            ks2 = (k1 ^ k2) ^ _const_i32(_GOLDEN)
            sg = step * js_per_step + js
            sbase = pb * (plane * n_row) + sg * jsub
            x1 = lane_tile + (k2 + riota + sbase)
            x0, x1 = _threefry_pair(k1, k2, ks2, k1, x1)
            mant = lax.shift_right_logical(x0 ^ x1, jnp.int32(9))
            for c in range(3):
                bm = carry[2 * c]
                bs = carry[2 * c + 1]
                mvld = jnp.where(m == c, mant, jnp.int32(-1))
                upd = mvld > bm
                carry[2 * c] = jnp.where(upd, mvld, bm)
                carry[2 * c + 1] = jnp.where(upd, sg, bs)
            return tuple(carry)

        carry = []
        for c in range(3):
            carry.append(bm_ref[pl.ds(c * jsub, jsub),
                                pl.ds(pb * plane, plane)])
            carry.append(bs_ref[pl.ds(c * jsub, jsub),
                                pl.ds(pb * plane, plane)])
        carry = lax.fori_loop(0, js_per_step, body, tuple(carry), unroll=8)
        for c in range(3):
            bm_ref[pl.ds(c * jsub, jsub), pl.ds(pb * plane, plane)] = \
                carry[2 * c]
            bs_ref[pl.ds(c * jsub, jsub), pl.ds(pb * plane, plane)] = \
                carry[2 * c + 1]

    @pl.when(step == nsteps - 1)
    def _finalize():
        rio = lax.broadcasted_iota(jnp.int32, (jsub, plane), 0)
        for pb in range(pblocks):
            for c in range(3):
                am = bm_ref[pl.ds(c * jsub, jsub), pl.ds(pb * plane, plane)]
                asg = bs_ref[pl.ds(c * jsub, jsub), pl.ds(pb * plane, plane)]
                ar = rio
                rows = jsub
                while rows > 1:
                    rows //= 2
                    bm2 = am[rows:2 * rows]
                    bsg = asg[rows:2 * rows]
                    br = ar[rows:2 * rows]
                    am, asg, ar = am[:rows], asg[:rows], ar[:rows]
                    tk = (bm2 > am) | ((bm2 == am) & (
                        (bsg < asg) | ((bsg == asg) & (br < ar))))
                    am = jnp.where(tk, bm2, am)
                    asg = jnp.where(tk, bsg, asg)
                    ar = jnp.where(tk, br, ar)
                sl = (pl.ds(c, 1), pl.ds(pb * plane, plane))
                out_ref[sl] = asg * jnp.int32(jsub) + ar
                mant_ref[sl] = am


def _sample_indices(mask_head, j_total, num_pairs, n_row=None, jsub=16,
                    js_per_step=128, plane=512, interpret=False):
    if n_row is None:
        n_row = j_total
    nsteps = j_total // (jsub * js_per_step)
    pblocks = num_pairs // plane
    mask_t = mask_head.reshape(j_total // jsub, jsub).T
    body = functools.partial(_sample_body, n_row, nsteps, js_per_step, jsub,
                             pblocks, plane)
    out, mant = pl.pallas_call(
        body,
        grid=(nsteps,),
        in_specs=[pl.BlockSpec((jsub, js_per_step), lambda s: (0, s))],
        out_specs=[pl.BlockSpec((8, num_pairs), lambda s: (0, 0)),
                   pl.BlockSpec((8, num_pairs), lambda s: (0, 0))],
        out_shape=[jax.ShapeDtypeStruct((8, num_pairs), jnp.int32),
                   jax.ShapeDtypeStruct((8, num_pairs), jnp.int32)],
        scratch_shapes=[pltpu.VMEM((3 * jsub, num_pairs), jnp.int32),
                        pltpu.VMEM((3 * jsub, num_pairs), jnp.int32)],
        interpret=interpret,
    )(mask_t)
    return out, mant


_J_TC = 186368


def _sc_sample(mask_hbm, n_total, j0, nj, num_p):
    info = plsc.get_sparse_core_info()
    nc, ns = info.num_cores, info.num_subcores
    nw = nc * ns
    ppw = num_p // nw
    jc = 2048
    nch = nj // jc
    nvec = jc // 16
    mesh = plsc.VectorSubcoreMesh(core_axis_name="c", subcore_axis_name="s")

    kb1 = _const_i32(_KEYS[0][0])
    kv1 = _const_i32(_KEYS[1][0])
    ka1 = _const_i32(_KEYS[2][0])
    kb2 = _const_i32(_KEYS[0][1])
    kv2 = _const_i32(_KEYS[1][1])
    ka2 = _const_i32(_KEYS[2][1])

    @functools.partial(
        pl.kernel,
        mesh=mesh,
        compiler_params=pltpu.CompilerParams(use_tc_tiling_on_sc=False),
        out_type=jax.ShapeDtypeStruct((nw, 6 * ppw), jnp.int32),
        scratch_types=[
            pltpu.VMEM((jc,), jnp.int32),
            pltpu.VMEM((jc,), jnp.int32),
            pltpu.VMEM((jc,), jnp.int32),
            pltpu.VMEM((jc,), jnp.int32),
            pltpu.VMEM((ppw, 6, 16), jnp.int32),
            pltpu.VMEM((6 * ppw,), jnp.int32),
        ],
    )
    def k(mask_ref, out_hbm, mv, k1a, k2a, ks2a, st, outv):
        wid = lax.axis_index("s") * nc + lax.axis_index("c")
        pw0 = wid * ppw
        lanes = lax.iota(jnp.int32, 16)
        neg1 = jnp.full((16,), -1, jnp.int32)
        zero = jnp.zeros((16,), jnp.int32)

        def initp(p, _):
            for c in range(3):
                st[p, c, :] = neg1
                st[p, 3 + c, :] = zero
            return 0

        lax.fori_loop(0, ppw, initp, 0)

        def chunk_body(ch, _):
            pltpu.sync_copy(
                mask_ref.at[pl.ds(pl.multiple_of(j0 + ch * jc, jc), jc)], mv)

            def keys_body(jv, _):
                off = pl.ds(pl.multiple_of(jv * 16, 16), 16)
                m = mv[off]
                k1a[off] = jnp.where(m == 0, kb1, jnp.where(m == 1, kv1, ka1))
                k2a[off] = jnp.where(m == 0, kb2, jnp.where(m == 1, kv2, ka2))
                ks2a[off] = (k1a[off] ^ k2a[off]) ^ _const_i32(_GOLDEN)
                return 0

            lax.fori_loop(0, nvec, keys_body, 0)

            def pq_body(pq, _):
                state = []
                for q in range(4):
                    for r in range(6):
                        state.append(st[pq * 4 + q, r, :])

                def jv_body(jv, carry):
                    carry = list(carry)
                    off = pl.ds(pl.multiple_of(jv * 16, 16), 16)
                    m = mv[off]
                    k1v = k1a[off]
                    k2v = k2a[off]
                    ks2v = ks2a[off]
                    v0 = m == 0
                    v1 = m == 1
                    v2 = m == 2
                    jvg = j0 // 16 + ch * nvec + jv
                    for q in range(4):
                        pg = pw0 + pq * 4 + q
                        base = pg * n_total + j0 + ch * jc + jv * 16
                        x1 = k2v + (lanes + base)
                        x0 = k1v
                        x0, x1 = _threefry_pair(k1v, k2v, ks2v, x0, x1)
                        mant = lax.shift_right_logical(x0 ^ x1, jnp.int32(9))
                        for c, vc in enumerate((v0, v1, v2)):
                            bm = carry[q * 6 + c]
                            bj = carry[q * 6 + 3 + c]
                            mvld = jnp.where(vc, mant, jnp.int32(-1))
                            upd = mvld > bm
                            carry[q * 6 + c] = jnp.where(upd, mvld, bm)
                            carry[q * 6 + 3 + c] = jnp.where(upd, jvg, bj)
                    return tuple(carry)

                state = lax.fori_loop(0, nvec, jv_body, tuple(state))
                for q in range(4):
                    for r in range(6):
                        st[pq * 4 + q, r, :] = state[q * 6 + r]
                return 0

            lax.fori_loop(0, ppw // 4, pq_body, 0)
            return 0

        lax.fori_loop(0, nch, chunk_body, 0)

        big = jnp.int32(0x7FFFFFFF)

        def fin_group(g, _):
            for c in range(3):
                acc = zero
                accm = zero
                for t in range(16):
                    p = g * 16 + t
                    best = jnp.int32(-1)
                    bestj = big
                    bmv = st[p, c, :]
                    bjv = st[p, 3 + c, :]
                    for l in range(16):
                        m_l = bmv[l]
                        j_l = bjv[l] * 16 + l
                        take = (m_l > best) | ((m_l == best) & (j_l < bestj))
                        best = jnp.where(take, m_l, best)
                        bestj = jnp.where(take, j_l, bestj)
                    acc = jnp.where(lanes == t, bestj, acc)
                    accm = jnp.where(lanes == t, best, accm)
                outv[pl.ds(pl.multiple_of(c * ppw + g * 16, 16), 16)] = acc
                outv[pl.ds(pl.multiple_of(3 * ppw + c * ppw + g * 16, 16),
                           16)] = accm
            return 0

        lax.fori_loop(0, ppw // 16, fin_group, 0)
        pltpu.sync_copy(outv, out_hbm.at[wid])

    out = k(mask_hbm)
    o = out.reshape(nw, 2, 3, ppw)
    idx = o[:, 0].transpose(1, 0, 2).reshape(3, num_p)
    mant = o[:, 1].transpose(1, 0, 2).reshape(3, num_p)
    return idx, mant


def _sc_gather_loss(ft, idx_tc, mant_tc, idx_sc, mant_sc):
    info = plsc.get_sparse_core_info()
    nc, ns = info.num_cores, info.num_subcores
    nw = nc * ns
    bpw = NUM_POS // nw
    d = ft.shape[1]
    mesh = plsc.VectorSubcoreMesh(core_axis_name="c", subcore_axis_name="s")

    @functools.partial(
        pl.kernel,
        mesh=mesh,
        compiler_params=pltpu.CompilerParams(use_tc_tiling_on_sc=False),
        out_type=jax.ShapeDtypeStruct((nw, 16), jnp.float32),
        scratch_types=[
            pltpu.VMEM((bpw,), jnp.int32),
            pltpu.VMEM((bpw,), jnp.int32),
            pltpu.VMEM((bpw,), jnp.int32),
            pltpu.VMEM((bpw,), jnp.int32),
            pltpu.VMEM((bpw,), jnp.int32),
            pltpu.VMEM((bpw,), jnp.int32),
            pltpu.VMEM((bpw, d), jnp.float32),
            pltpu.VMEM((bpw, d), jnp.float32),
            pltpu.VMEM((bpw, d), jnp.float32),
            pltpu.VMEM((16,), jnp.float32),
            pltpu.SemaphoreType.DMA,
            pltpu.SemaphoreType.DMA,
            pltpu.SemaphoreType.DMA,
        ],
    )
    def k(ft_hbm, itc_hbm, mtc_hbm, isc_hbm, msc_hbm, out_hbm,
          i0, i1, i2, sidx, smt, sms, r0, r1, r2, acc_v, s0, s1, s2):
        wid = lax.axis_index("s") * nc + lax.axis_index("c")
        base = wid * bpw
        idx_bufs = (i0, i1, i2)
        row_bufs = (r0, r1, r2)
        sems = (s0, s1, s2)
        copies = []
        for c in range(3):
            sl = pl.ds(c * NUM_POS + base, bpw)
            pltpu.sync_copy(itc_hbm.at[sl], idx_bufs[c])
            pltpu.sync_copy(mtc_hbm.at[sl], smt)
            pltpu.sync_copy(isc_hbm.at[sl], sidx)
            pltpu.sync_copy(msc_hbm.at[sl], sms)
            for g in range(bpw // 16):
                off = pl.ds(g * 16, 16)
                take_sc = sms[off] > smt[off]
                idx_bufs[c][off] = jnp.where(take_sc, sidx[off],
                                             idx_bufs[c][off])
            cp = pltpu.async_copy(ft_hbm.at[idx_bufs[c]], row_bufs[c], sems[c])
            copies.append(cp)
        for cp in copies:
            cp.wait()

        half = jnp.float32(0.5)
        one = jnp.float32(1.0)

        def phi(dv):
            ad = jnp.abs(dv)
            return jnp.where(ad < one, half * dv * dv, ad - half)

        def srow(s, acc):
            for cc in range(d // 16):
                sl = pl.ds(cc * 16, 16)
                a = r0[s, sl]
                b = r1[s, sl]
                e = r2[s, sl]
                acc = acc + phi(a - b) + phi(b - e) + phi(a - e)
            return acc

        acc = lax.fori_loop(0, bpw, srow, jnp.zeros((16,), jnp.float32))
        acc_v[...] = acc
        pltpu.sync_copy(acc_v, out_hbm.at[wid])

    return k(ft, idx_tc, mant_tc, idx_sc, mant_sc)


def kernel(features_flat, mask_flat):
    n_total = mask_flat.shape[0]
    mask_i32 = mask_flat.astype(jnp.int32)
    idx_tc, mant_tc = _sample_indices(mask_i32[:_J_TC], _J_TC, NUM_POS,
                                      n_row=n_total)
    idx_sc, mant_sc = _sc_sample(mask_i32, n_total, _J_TC, n_total - _J_TC,
                                 NUM_POS)
    ft = features_flat.T
    parts = _sc_gather_loss(ft, idx_tc[:3].reshape(-1),
                            mant_tc[:3].reshape(-1), idx_sc.reshape(-1),
                            mant_sc.reshape(-1))
    denom = jnp.float32(features_flat.shape[0] * NUM_POS)
    return jnp.sum(parts) / denom
